# speedup vs baseline: 234.4050x; 234.4050x over previous
import functools

import jax
import jax.numpy as jnp
from jax import lax
from jax.experimental import pallas as pl
from jax.experimental.pallas import tpu as pltpu
from jax.experimental.pallas import tpu_sc as plsc

_NUM_NODES = 1_100_000
_NUM_PHYS = 1_000_000
_NUM_PINS = 4_000_000

_NC = 2
_NS = 16
_NW = _NC * _NS
_PINS_PER_W = _NUM_PINS // _NW
_CHUNK = 5000
_CHUNKS_PER_W = _PINS_PER_W // _CHUNK
_CBUF = 5008
_GROUPS = _CBUF // 16


@functools.partial(
    pl.kernel,
    out_type=jax.ShapeDtypeStruct((2 * _NUM_PINS,), jnp.float32),
    mesh=plsc.VectorSubcoreMesh(core_axis_name="c", subcore_axis_name="s"),
    scratch_types=[
        pltpu.VMEM((_CBUF,), jnp.int32),
        pltpu.VMEM((_CBUF,), jnp.float32),
        pltpu.VMEM((_CBUF,), jnp.float32),
        pltpu.VMEM((_CBUF,), jnp.float32),
        pltpu.VMEM((_CBUF,), jnp.float32),
        pltpu.SemaphoreType.DMA,
        pltpu.SemaphoreType.DMA,
    ],
)
def _pinpos_sc(pos_hbm, posy_hbm, offx_hbm, offy_hbm, map_hbm, out_hbm,
               idx_v, gx_v, gy_v, ox_v, oy_v, semx, semy):
    wid = lax.axis_index("s") * _NC + lax.axis_index("c")
    base = wid * _PINS_PER_W
    idx_v[pl.ds(_CBUF - 16, 16)] = jnp.zeros((16,), jnp.int32)

    def chunk_body(j, carry):
        off = pl.multiple_of(base + j * _CHUNK, 8)
        pltpu.sync_copy(map_hbm.at[pl.ds(off, _CHUNK)], idx_v.at[pl.ds(0, _CHUNK)])
        cx = pltpu.async_copy(pos_hbm.at[idx_v], gx_v, semx)
        cy = pltpu.async_copy(posy_hbm.at[idx_v], gy_v, semy)
        pltpu.sync_copy(offx_hbm.at[pl.ds(off, _CHUNK)], ox_v.at[pl.ds(0, _CHUNK)])
        pltpu.sync_copy(offy_hbm.at[pl.ds(off, _CHUNK)], oy_v.at[pl.ds(0, _CHUNK)])
        cx.wait()
        cy.wait()

        def add_body(i, c):
            s = pl.ds(i * 16, 16)
            gx_v[s] = gx_v[s] + ox_v[s]
            gy_v[s] = gy_v[s] + oy_v[s]
            return c

        lax.fori_loop(0, _GROUPS, add_body, 0)
        pltpu.sync_copy(gx_v.at[pl.ds(0, _CHUNK)], out_hbm.at[pl.ds(off, _CHUNK)])
        pltpu.sync_copy(gy_v.at[pl.ds(0, _CHUNK)],
                        out_hbm.at[pl.ds(_NUM_PINS + off, _CHUNK)])
        return carry

    lax.fori_loop(0, _CHUNKS_PER_W, chunk_body, 0)


def kernel(pos, pin_offset_x, pin_offset_y, pin2node_map):
    posy = lax.slice(pos, (_NUM_NODES,), (_NUM_NODES + _NUM_PHYS,))
    return _pinpos_sc(pos, posy, pin_offset_x, pin_offset_y, pin2node_map)
